# Initial kernel scaffold; baseline (speedup 1.0000x reference)
import jax
import jax.numpy as jnp
from jax.experimental import pallas as pl


def kernel(x, edge_index, W0_1, b0_1, W1_1, b1_1, W2_1, b2_1, W0_2, b0_2, W1_2, b1_2):
    raise NotImplementedError("write your pallas kernel here")


# speedup vs baseline: 25.7294x; 25.7294x over previous
import functools

import jax
import jax.numpy as jnp
from jax import lax
from jax.experimental import pallas as pl
from jax.experimental.pallas import tpu as pltpu
from jax.experimental.pallas import tpu_sc as plsc

N_NODES = 10000
N_PAD = 10240
NC = 2
NS = 16
NW = NC * NS
K_EDGE = 125
NCHUNK = 80
ROWS_SUB = N_PAD // NS
ZCH = 64

_MESH = plsc.VectorSubcoreMesh(core_axis_name="c", subcore_axis_name="s")


def _vfill(buf, rows, val):
    def body(i, carry):
        for j in range(8):
            buf[i, pl.ds(j * 16, 16)] = jnp.full((16,), val, jnp.float32)
        return carry

    lax.fori_loop(0, rows, body, 0)


def _zero_stripe(acc, zbuf, s):
    _vfill(zbuf, ZCH, 0.0)
    for z in range(ROWS_SUB // ZCH):
        pltpu.sync_copy(zbuf, acc.at[pl.ds(s * ROWS_SUB + z * ZCH, ZCH)])


def _sc_degree_body(col_hbm, out_hbm, idx_v, ones_v, zbuf, acc):
    c = lax.axis_index("c")
    s = lax.axis_index("s")
    wid = c * NS + s
    _zero_stripe(acc, zbuf, s)
    _vfill(ones_v, K_EDGE, 1.0)
    pltpu.sync_copy(col_hbm.at[wid], idx_v)
    plsc.subcore_barrier()

    def chunk(j, carry):
        pltpu.sync_copy(ones_v, acc.at[idx_v.at[j]], add=True)
        return carry

    lax.fori_loop(0, NCHUNK, chunk, 0)
    plsc.subcore_barrier()
    pltpu.sync_copy(acc.at[pl.ds(s * ROWS_SUB, ROWS_SUB)],
                    out_hbm.at[c, pl.ds(s * ROWS_SUB, ROWS_SUB)])


def _sc_scatter_body(vals_hbm, row_hbm, col_hbm, out_hbm,
                     ridx_v, cidx_v, gbuf, zbuf, acc, sem):
    c = lax.axis_index("c")
    s = lax.axis_index("s")
    wid = c * NS + s
    _zero_stripe(acc, zbuf, s)
    pltpu.sync_copy(row_hbm.at[wid], ridx_v)
    pltpu.sync_copy(col_hbm.at[wid], cidx_v)
    plsc.subcore_barrier()

    def chunk(j, carry):
        pltpu.async_copy(vals_hbm.at[ridx_v.at[j]], gbuf, sem).wait()
        pltpu.sync_copy(gbuf, acc.at[cidx_v.at[j]], add=True)
        return carry

    lax.fori_loop(0, NCHUNK, chunk, 0)
    plsc.subcore_barrier()
    pltpu.sync_copy(acc.at[pl.ds(s * ROWS_SUB, ROWS_SUB)],
                    out_hbm.at[c, pl.ds(s * ROWS_SUB, ROWS_SUB)])


def _sc_degree(col3):
    call = pl.kernel(
        _sc_degree_body,
        out_type=jax.ShapeDtypeStruct((NC, N_PAD, 128), jnp.float32),
        mesh=_MESH,
        scratch_types=[
            pltpu.VMEM((NCHUNK, K_EDGE), jnp.int32),
            pltpu.VMEM((K_EDGE, 128), jnp.float32),
            pltpu.VMEM((ZCH, 128), jnp.float32),
            pltpu.VMEM_SHARED((N_PAD, 128), jnp.float32),
        ],
    )
    return call(col3)


def _sc_scatter(vals, row3, col3, width):
    call = pl.kernel(
        _sc_scatter_body,
        out_type=jax.ShapeDtypeStruct((NC, N_PAD, width), jnp.float32),
        mesh=_MESH,
        scratch_types=[
            pltpu.VMEM((NCHUNK, K_EDGE), jnp.int32),
            pltpu.VMEM((NCHUNK, K_EDGE), jnp.int32),
            pltpu.VMEM((K_EDGE, width), jnp.float32),
            pltpu.VMEM((ZCH, width), jnp.float32),
            pltpu.VMEM_SHARED((N_PAD, width), jnp.float32),
            pltpu.SemaphoreType.DMA,
        ],
    )
    return call(vals, row3, col3)


_RB = 400
_GRID = N_NODES // _RB


def _tc_prep_body(dp_ref, x_ref, u_ref, dis_ref):
    deg = dp_ref[0] + dp_ref[1] + 1.0
    disb = lax.rsqrt(deg)
    u_ref[...] = x_ref[...] * disb
    dis_ref[...] = disb


def _tc_mm_body(x_ref, w_ref, b_ref, o_ref):
    o_ref[...] = jnp.dot(x_ref[...], w_ref[...],
                         preferred_element_type=jnp.float32) + b_ref[...]


def _tc_mid_body(p_ref, u_ref, dis_ref, w_ref, b_ref, o1_ref, w2_ref):
    s1 = p_ref[0] + p_ref[1] + u_ref[...]
    dis = dis_ref[...]
    h1 = dis * s1
    w2_ref[...] = h1 * dis
    o1_ref[...] = jnp.dot(h1, w_ref[...],
                          preferred_element_type=jnp.float32) + b_ref[...]


def _tc_big_body(p_ref, w2_ref, dis_ref, out0_ref, o1_ref,
                 w21_ref, b21_ref, w02_ref, b02_ref, w12_ref,
                 z0_ref, u2_ref):
    s2 = p_ref[0] + p_ref[1] + w2_ref[...]
    dis = dis_ref[...]
    h2 = dis * s2
    o2 = jnp.dot(h2, w21_ref[...], preferred_element_type=jnp.float32) + b21_ref[...]
    xa = jax.nn.relu(out0_ref[...])
    xb = jax.nn.relu(o1_ref[...])
    xc = jax.nn.relu(o2)
    dot = functools.partial(jnp.dot, preferred_element_type=jnp.float32)
    z0 = (dot(xa, w02_ref[0:512]) + dot(xb, w02_ref[512:1024])
          + dot(xc, w02_ref[1024:1536]) + b02_ref[...])
    y1 = (dot(xa, w12_ref[0:512]) + dot(xb, w12_ref[512:1024])
          + dot(xc, w12_ref[1024:1536]))
    z0_ref[...] = z0
    u2 = dis[:, :40] * y1
    u2_ref[...] = jnp.concatenate([u2, jnp.zeros((_RB, 88), jnp.float32)], axis=1)


def _tc_final_body(p_ref, u2_ref, dis_ref, z0_ref, b12_ref, out_ref):
    s3 = p_ref[0] + p_ref[1] + u2_ref[...]
    h3 = dis_ref[...][:, :40] * s3[:, :40]
    o12 = h3 + b12_ref[...]
    logits = jnp.concatenate([z0_ref[...], o12], axis=1)
    m = jnp.max(logits, axis=1, keepdims=True)
    ex = jnp.exp(logits - m)
    se = jnp.sum(ex, axis=1, keepdims=True)
    out_ref[...] = logits - m - jnp.log(se)


def _row_spec(width):
    return pl.BlockSpec((_RB, width), lambda i: (i, 0))


def _part_spec(width):
    return pl.BlockSpec((NC, _RB, width), lambda i: (0, i, 0))


def _full_spec(shape):
    nd = len(shape)
    return pl.BlockSpec(shape, lambda i: (0,) * nd)


def kernel(x, edge_index, W0_1, b0_1, W1_1, b1_1, W2_1, b2_1,
           W0_2, b0_2, W1_2, b1_2):
    E = edge_index.shape[1]
    assert E == NW * NCHUNK * K_EDGE

    row3 = edge_index[0].reshape(NW, NCHUNK, K_EDGE)
    col3 = edge_index[1].reshape(NW, NCHUNK, K_EDGE)

    out0 = pl.pallas_call(
        _tc_mm_body,
        grid=(_GRID,),
        in_specs=[_row_spec(128), _full_spec((128, 512)), _full_spec((1, 512))],
        out_specs=_row_spec(512),
        out_shape=jax.ShapeDtypeStruct((N_NODES, 512), jnp.float32),
    )(x, W0_1, b0_1.reshape(1, 512))

    deg_parts = _sc_degree(col3)
    u, dis = pl.pallas_call(
        _tc_prep_body,
        grid=(_GRID,),
        in_specs=[_part_spec(128), _row_spec(128)],
        out_specs=[_row_spec(128), _row_spec(128)],
        out_shape=[jax.ShapeDtypeStruct((N_NODES, 128), jnp.float32),
                   jax.ShapeDtypeStruct((N_NODES, 128), jnp.float32)],
    )(deg_parts, x)

    p1 = _sc_scatter(u, row3, col3, 128)
    o1, w2 = pl.pallas_call(
        _tc_mid_body,
        grid=(_GRID,),
        in_specs=[_part_spec(128), _row_spec(128), _row_spec(128),
                  _full_spec((128, 512)), _full_spec((1, 512))],
        out_specs=[_row_spec(512), _row_spec(128)],
        out_shape=[jax.ShapeDtypeStruct((N_NODES, 512), jnp.float32),
                   jax.ShapeDtypeStruct((N_NODES, 128), jnp.float32)],
    )(p1, u, dis, W1_1, b1_1.reshape(1, 512))

    p2 = _sc_scatter(w2, row3, col3, 128)
    z0, u2 = pl.pallas_call(
        _tc_big_body,
        grid=(_GRID,),
        in_specs=[_part_spec(128), _row_spec(128), _row_spec(128),
                  _row_spec(512), _row_spec(512),
                  _full_spec((128, 512)), _full_spec((1, 512)),
                  _full_spec((1536, 40)), _full_spec((1, 40)),
                  _full_spec((1536, 40))],
        out_specs=[_row_spec(40), _row_spec(128)],
        out_shape=[jax.ShapeDtypeStruct((N_NODES, 40), jnp.float32),
                   jax.ShapeDtypeStruct((N_NODES, 128), jnp.float32)],
    )(p2, w2, dis, out0, o1, W2_1, b2_1.reshape(1, 512),
      W0_2, b0_2.reshape(1, 40), W1_2)

    p3 = _sc_scatter(u2, row3, col3, 128)
    out = pl.pallas_call(
        _tc_final_body,
        grid=(_GRID,),
        in_specs=[_part_spec(128), _row_spec(128), _row_spec(128),
                  _row_spec(40), _full_spec((1, 40))],
        out_specs=_row_spec(80),
        out_shape=jax.ShapeDtypeStruct((N_NODES, 80), jnp.float32),
    )(p3, u2, dis, z0, b1_2.reshape(1, 40))
    return out


# speedup vs baseline: 30.5597x; 1.1877x over previous
import functools

import jax
import jax.numpy as jnp
from jax import lax
from jax.experimental import pallas as pl
from jax.experimental.pallas import tpu as pltpu
from jax.experimental.pallas import tpu_sc as plsc

N_NODES = 10000
N_PAD = 10240
NC = 2
NS = 16
NW = NC * NS
K_EDGE = 125
NCHUNK = 80
HCHUNK = NCHUNK // 2
ROWS_SUB = N_PAD // NS
ZCH = 8
NBUF = 2
DWIN = 8

_MESH = plsc.VectorSubcoreMesh(core_axis_name="c", subcore_axis_name="s")


def _vfill(buf, rows, val):
    def body(i, carry):
        for j in range(8):
            buf[i, pl.ds(j * 16, 16)] = jnp.full((16,), val, jnp.float32)
        return carry

    lax.fori_loop(0, rows, body, 0)


def _zero_stripe(acc, zbuf, s):
    _vfill(zbuf, ZCH, 0.0)
    for z in range(ROWS_SUB // ZCH):
        pltpu.sync_copy(zbuf, acc.at[pl.ds(s * ROWS_SUB + z * ZCH, ZCH)])


def _sc_degree_body(col_hbm, out_hbm, idx_v, ones_v, zbuf, acc, sem):
    c = lax.axis_index("c")
    s = lax.axis_index("s")
    wid = c * NS + s
    _zero_stripe(acc, zbuf, s)
    _vfill(ones_v, K_EDGE, 1.0)
    pltpu.sync_copy(col_hbm.at[wid], idx_v)
    plsc.subcore_barrier()

    def chunk(j, carry):
        @pl.when(j >= DWIN)
        def _drain():
            pltpu.make_async_copy(ones_v, acc.at[idx_v.at[j - DWIN]], sem).wait()

        pltpu.async_copy(ones_v, acc.at[idx_v.at[j]], sem, add=True)
        return carry

    lax.fori_loop(0, NCHUNK, chunk, 0)

    def drain(j, carry):
        pltpu.make_async_copy(ones_v, acc.at[idx_v.at[j]], sem).wait()
        return carry

    lax.fori_loop(NCHUNK - DWIN, NCHUNK, drain, 0)
    plsc.subcore_barrier()
    pltpu.sync_copy(acc.at[pl.ds(s * ROWS_SUB, ROWS_SUB)],
                    out_hbm.at[c, pl.ds(s * ROWS_SUB, ROWS_SUB)])


def _sc_scatter_body(vals_hbm, row_hbm, col_hbm, out_hbm,
                     ridx_v, cidx_v, gbuf, zbuf, acc,
                     gs0, gs1, ss0, ss1):
    gsems = (gs0, gs1)
    ssems = (ss0, ss1)
    c = lax.axis_index("c")
    s = lax.axis_index("s")
    wid = c * NS + s
    _zero_stripe(acc, zbuf, s)
    plsc.subcore_barrier()

    for h in range(2):
        pltpu.sync_copy(row_hbm.at[wid, pl.ds(h * HCHUNK, HCHUNK)], ridx_v)
        pltpu.sync_copy(col_hbm.at[wid, pl.ds(h * HCHUNK, HCHUNK)], cidx_v)
        for b in range(NBUF):
            pltpu.async_copy(vals_hbm.at[ridx_v.at[b]], gbuf.at[b], gsems[b])

        def round_(r, carry):
            for b in range(NBUF):
                j = r * NBUF + b
                pltpu.make_async_copy(vals_hbm.at[ridx_v.at[j]], gbuf.at[b],
                                      gsems[b]).wait()
                pltpu.async_copy(gbuf.at[b], acc.at[cidx_v.at[j]], ssems[b],
                                 add=True)
                pb = (b - 1) % NBUF
                jprev = j - 1
                jnext = jprev + NBUF

                @pl.when(jnp.logical_and(jprev >= 0, jnext < HCHUNK))
                def _refill():
                    pltpu.make_async_copy(gbuf.at[pb],
                                          acc.at[cidx_v.at[jprev]],
                                          ssems[pb]).wait()
                    pltpu.async_copy(vals_hbm.at[ridx_v.at[jnext]],
                                     gbuf.at[pb], gsems[pb])
            return carry

        lax.fori_loop(0, HCHUNK // NBUF, round_, 0)
        for b in range(NBUF):
            j = HCHUNK - NBUF + b
            pltpu.make_async_copy(gbuf.at[b], acc.at[cidx_v.at[j]],
                                  ssems[b]).wait()
    plsc.subcore_barrier()
    pltpu.sync_copy(acc.at[pl.ds(s * ROWS_SUB, ROWS_SUB)],
                    out_hbm.at[c, pl.ds(s * ROWS_SUB, ROWS_SUB)])


def _sc_degree(col3):
    call = pl.kernel(
        _sc_degree_body,
        out_type=jax.ShapeDtypeStruct((NC, N_PAD, 128), jnp.float32),
        mesh=_MESH,
        scratch_types=[
            pltpu.VMEM((NCHUNK, K_EDGE), jnp.int32),
            pltpu.VMEM((K_EDGE, 128), jnp.float32),
            pltpu.VMEM((ZCH, 128), jnp.float32),
            pltpu.VMEM_SHARED((N_PAD, 128), jnp.float32),
            pltpu.SemaphoreType.DMA,
        ],
    )
    return call(col3)


def _sc_scatter(vals, row3, col3, width):
    call = pl.kernel(
        _sc_scatter_body,
        out_type=jax.ShapeDtypeStruct((NC, N_PAD, width), jnp.float32),
        mesh=_MESH,
        scratch_types=[
            pltpu.VMEM((HCHUNK, K_EDGE), jnp.int32),
            pltpu.VMEM((HCHUNK, K_EDGE), jnp.int32),
            pltpu.VMEM((NBUF, K_EDGE, width), jnp.float32),
            pltpu.VMEM((ZCH, width), jnp.float32),
            pltpu.VMEM_SHARED((N_PAD, width), jnp.float32),
        ] + [pltpu.SemaphoreType.DMA] * (2 * NBUF),
    )
    return call(vals, row3, col3)


_RB = 400
_GRID = N_NODES // _RB


def _tc_prep_body(dp_ref, x_ref, u_ref, dis_ref):
    deg = dp_ref[0] + dp_ref[1] + 1.0
    disb = lax.rsqrt(deg)
    u_ref[...] = x_ref[...] * disb
    dis_ref[...] = disb


def _tc_mm_body(x_ref, w_ref, b_ref, o_ref):
    o_ref[...] = jnp.dot(x_ref[...], w_ref[...],
                         preferred_element_type=jnp.float32) + b_ref[...]


def _tc_mid_body(p_ref, u_ref, dis_ref, w_ref, b_ref, o1_ref, w2_ref):
    s1 = p_ref[0] + p_ref[1] + u_ref[...]
    dis = dis_ref[...]
    h1 = dis * s1
    w2_ref[...] = h1 * dis
    o1_ref[...] = jnp.dot(h1, w_ref[...],
                          preferred_element_type=jnp.float32) + b_ref[...]


def _tc_big_body(p_ref, w2_ref, dis_ref, out0_ref, o1_ref,
                 w21_ref, b21_ref, w02_ref, b02_ref, w12_ref,
                 z0_ref, u2_ref):
    s2 = p_ref[0] + p_ref[1] + w2_ref[...]
    dis = dis_ref[...]
    h2 = dis * s2
    o2 = jnp.dot(h2, w21_ref[...], preferred_element_type=jnp.float32) + b21_ref[...]
    xa = jax.nn.relu(out0_ref[...])
    xb = jax.nn.relu(o1_ref[...])
    xc = jax.nn.relu(o2)
    dot = functools.partial(jnp.dot, preferred_element_type=jnp.float32)
    z0 = (dot(xa, w02_ref[0:512]) + dot(xb, w02_ref[512:1024])
          + dot(xc, w02_ref[1024:1536]) + b02_ref[...])
    y1 = (dot(xa, w12_ref[0:512]) + dot(xb, w12_ref[512:1024])
          + dot(xc, w12_ref[1024:1536]))
    z0_ref[...] = z0
    u2 = dis[:, :40] * y1
    u2_ref[...] = jnp.concatenate([u2, jnp.zeros((_RB, 88), jnp.float32)], axis=1)


def _tc_final_body(p_ref, u2_ref, dis_ref, z0_ref, b12_ref, out_ref):
    s3 = p_ref[0] + p_ref[1] + u2_ref[...]
    h3 = dis_ref[...][:, :40] * s3[:, :40]
    o12 = h3 + b12_ref[...]
    logits = jnp.concatenate([z0_ref[...], o12], axis=1)
    m = jnp.max(logits, axis=1, keepdims=True)
    ex = jnp.exp(logits - m)
    se = jnp.sum(ex, axis=1, keepdims=True)
    out_ref[...] = logits - m - jnp.log(se)


def _row_spec(width):
    return pl.BlockSpec((_RB, width), lambda i: (i, 0))


def _part_spec(width):
    return pl.BlockSpec((NC, _RB, width), lambda i: (0, i, 0))


def _full_spec(shape):
    nd = len(shape)
    return pl.BlockSpec(shape, lambda i: (0,) * nd)


def kernel(x, edge_index, W0_1, b0_1, W1_1, b1_1, W2_1, b2_1,
           W0_2, b0_2, W1_2, b1_2):
    E = edge_index.shape[1]
    assert E == NW * NCHUNK * K_EDGE

    row3 = edge_index[0].reshape(NW, NCHUNK, K_EDGE)
    col3 = edge_index[1].reshape(NW, NCHUNK, K_EDGE)

    out0 = pl.pallas_call(
        _tc_mm_body,
        grid=(_GRID,),
        in_specs=[_row_spec(128), _full_spec((128, 512)), _full_spec((1, 512))],
        out_specs=_row_spec(512),
        out_shape=jax.ShapeDtypeStruct((N_NODES, 512), jnp.float32),
    )(x, W0_1, b0_1.reshape(1, 512))

    deg_parts = _sc_degree(col3)
    u, dis = pl.pallas_call(
        _tc_prep_body,
        grid=(_GRID,),
        in_specs=[_part_spec(128), _row_spec(128)],
        out_specs=[_row_spec(128), _row_spec(128)],
        out_shape=[jax.ShapeDtypeStruct((N_NODES, 128), jnp.float32),
                   jax.ShapeDtypeStruct((N_NODES, 128), jnp.float32)],
    )(deg_parts, x)

    p1 = _sc_scatter(u, row3, col3, 128)
    o1, w2 = pl.pallas_call(
        _tc_mid_body,
        grid=(_GRID,),
        in_specs=[_part_spec(128), _row_spec(128), _row_spec(128),
                  _full_spec((128, 512)), _full_spec((1, 512))],
        out_specs=[_row_spec(512), _row_spec(128)],
        out_shape=[jax.ShapeDtypeStruct((N_NODES, 512), jnp.float32),
                   jax.ShapeDtypeStruct((N_NODES, 128), jnp.float32)],
    )(p1, u, dis, W1_1, b1_1.reshape(1, 512))

    p2 = _sc_scatter(w2, row3, col3, 128)
    z0, u2 = pl.pallas_call(
        _tc_big_body,
        grid=(_GRID,),
        in_specs=[_part_spec(128), _row_spec(128), _row_spec(128),
                  _row_spec(512), _row_spec(512),
                  _full_spec((128, 512)), _full_spec((1, 512)),
                  _full_spec((1536, 40)), _full_spec((1, 40)),
                  _full_spec((1536, 40))],
        out_specs=[_row_spec(40), _row_spec(128)],
        out_shape=[jax.ShapeDtypeStruct((N_NODES, 40), jnp.float32),
                   jax.ShapeDtypeStruct((N_NODES, 128), jnp.float32)],
    )(p2, w2, dis, out0, o1, W2_1, b2_1.reshape(1, 512),
      W0_2, b0_2.reshape(1, 40), W1_2)

    p3 = _sc_scatter(u2, row3, col3, 128)
    out = pl.pallas_call(
        _tc_final_body,
        grid=(_GRID,),
        in_specs=[_part_spec(128), _row_spec(128), _row_spec(128),
                  _row_spec(40), _full_spec((1, 40))],
        out_specs=_row_spec(80),
        out_shape=jax.ShapeDtypeStruct((N_NODES, 80), jnp.float32),
    )(p3, u2, dis, z0, b1_2.reshape(1, 40))
    return out


# speedup vs baseline: 30.9778x; 1.0137x over previous
import functools

import jax
import jax.numpy as jnp
from jax import lax
from jax.experimental import pallas as pl
from jax.experimental.pallas import tpu as pltpu
from jax.experimental.pallas import tpu_sc as plsc

N_NODES = 10000
N_PAD = 10240
NC = 2
NS = 16
NW = NC * NS
K_EDGE = 125
NCHUNK = 80
HCHUNK = NCHUNK // 2
ROWS_SUB = N_PAD // NS
ZCH = 32
NBUF = 2
DWIN = 8

_MESH = plsc.VectorSubcoreMesh(core_axis_name="c", subcore_axis_name="s")


def _vfill(buf, rows, width, val):
    def body(i, carry):
        for j in range(width // 16):
            buf[i, pl.ds(j * 16, 16)] = jnp.full((16,), val, jnp.float32)
        return carry

    lax.fori_loop(0, rows, body, 0)


def _zero_stripe(acc, zbuf, s, width):
    _vfill(zbuf, ZCH, width, 0.0)
    for z in range(ROWS_SUB // ZCH):
        pltpu.sync_copy(zbuf, acc.at[pl.ds(s * ROWS_SUB + z * ZCH, ZCH)])


def _sc_degree_body(col_hbm, out_hbm, idx_v, ones_v, zbuf, acc, sem):
    c = lax.axis_index("c")
    s = lax.axis_index("s")
    wid = c * NS + s
    _zero_stripe(acc, zbuf, s, 128)
    _vfill(ones_v, K_EDGE, 128, 1.0)
    pltpu.sync_copy(col_hbm.at[wid], idx_v)
    plsc.subcore_barrier()

    def chunk(j, carry):
        @pl.when(j >= DWIN)
        def _drain():
            pltpu.make_async_copy(ones_v, acc.at[idx_v.at[j - DWIN]], sem).wait()

        pltpu.async_copy(ones_v, acc.at[idx_v.at[j]], sem, add=True)
        return carry

    lax.fori_loop(0, NCHUNK, chunk, 0)

    def drain(j, carry):
        pltpu.make_async_copy(ones_v, acc.at[idx_v.at[j]], sem).wait()
        return carry

    lax.fori_loop(NCHUNK - DWIN, NCHUNK, drain, 0)
    plsc.subcore_barrier()
    pltpu.sync_copy(acc.at[pl.ds(s * ROWS_SUB, ROWS_SUB)],
                    out_hbm.at[c, pl.ds(s * ROWS_SUB, ROWS_SUB)])


def _sc_scatter_body(vals_hbm, row_hbm, col_hbm, out_hbm,
                     ridx_v, cidx_v, gbuf, zbuf, acc,
                     gs0, gs1, ss0, ss1):
    gsems = (gs0, gs1)
    ssems = (ss0, ss1)
    c = lax.axis_index("c")
    s = lax.axis_index("s")
    wid = c * NS + s
    _zero_stripe(acc, zbuf, s, acc.shape[1])
    plsc.subcore_barrier()

    for h in range(2):
        pltpu.sync_copy(row_hbm.at[wid, pl.ds(h * HCHUNK, HCHUNK)], ridx_v)
        pltpu.sync_copy(col_hbm.at[wid, pl.ds(h * HCHUNK, HCHUNK)], cidx_v)
        for b in range(NBUF):
            pltpu.async_copy(vals_hbm.at[ridx_v.at[b]], gbuf.at[b], gsems[b])

        def round_(r, carry):
            for b in range(NBUF):
                j = r * NBUF + b
                pltpu.make_async_copy(vals_hbm.at[ridx_v.at[j]], gbuf.at[b],
                                      gsems[b]).wait()
                pltpu.async_copy(gbuf.at[b], acc.at[cidx_v.at[j]], ssems[b],
                                 add=True)
                pb = (b - 1) % NBUF
                jprev = j - 1
                jnext = jprev + NBUF

                @pl.when(jnp.logical_and(jprev >= 0, jnext < HCHUNK))
                def _refill():
                    pltpu.make_async_copy(gbuf.at[pb],
                                          acc.at[cidx_v.at[jprev]],
                                          ssems[pb]).wait()
                    pltpu.async_copy(vals_hbm.at[ridx_v.at[jnext]],
                                     gbuf.at[pb], gsems[pb])
            return carry

        lax.fori_loop(0, HCHUNK // NBUF, round_, 0)
        for b in range(NBUF):
            j = HCHUNK - NBUF + b
            pltpu.make_async_copy(gbuf.at[b], acc.at[cidx_v.at[j]],
                                  ssems[b]).wait()
    plsc.subcore_barrier()
    pltpu.sync_copy(acc.at[pl.ds(s * ROWS_SUB, ROWS_SUB)],
                    out_hbm.at[c, pl.ds(s * ROWS_SUB, ROWS_SUB)])


def _sc_degree(col3):
    call = pl.kernel(
        _sc_degree_body,
        out_type=jax.ShapeDtypeStruct((NC, N_PAD, 128), jnp.float32),
        mesh=_MESH,
        scratch_types=[
            pltpu.VMEM((NCHUNK, K_EDGE), jnp.int32),
            pltpu.VMEM((K_EDGE, 128), jnp.float32),
            pltpu.VMEM((ZCH, 128), jnp.float32),
            pltpu.VMEM_SHARED((N_PAD, 128), jnp.float32),
            pltpu.SemaphoreType.DMA,
        ],
    )
    return call(col3)


def _sc_scatter(vals, row3, col3, width):
    call = pl.kernel(
        _sc_scatter_body,
        out_type=jax.ShapeDtypeStruct((NC, N_PAD, width), jnp.float32),
        mesh=_MESH,
        scratch_types=[
            pltpu.VMEM((HCHUNK, K_EDGE), jnp.int32),
            pltpu.VMEM((HCHUNK, K_EDGE), jnp.int32),
            pltpu.VMEM((NBUF, K_EDGE, width), jnp.float32),
            pltpu.VMEM((ZCH, width), jnp.float32),
            pltpu.VMEM_SHARED((N_PAD, width), jnp.float32),
        ] + [pltpu.SemaphoreType.DMA] * (2 * NBUF),
    )
    return call(vals, row3, col3)


_RB = 400
_GRID = N_NODES // _RB


def _tc_prep_body(dp_ref, x_ref, w_ref, b_ref, u_ref, dis_ref, out0_ref):
    deg = dp_ref[0] + dp_ref[1] + 1.0
    disb = lax.rsqrt(deg)
    u_ref[...] = x_ref[...] * disb
    dis_ref[...] = disb
    out0_ref[...] = jnp.dot(x_ref[...], w_ref[...],
                            preferred_element_type=jnp.float32) + b_ref[...]


def _tc_mid_body(p_ref, u_ref, dis_ref, w_ref, b_ref, o1_ref, w2_ref):
    s1 = p_ref[0] + p_ref[1] + u_ref[...]
    dis = dis_ref[...]
    h1 = dis * s1
    w2_ref[...] = h1 * dis
    o1_ref[...] = jnp.dot(h1, w_ref[...],
                          preferred_element_type=jnp.float32) + b_ref[...]


def _tc_big_body(p_ref, w2_ref, dis_ref, out0_ref, o1_ref,
                 w21_ref, b21_ref, w02_ref, b02_ref, w12_ref,
                 z0_ref, u2_ref):
    s2 = p_ref[0] + p_ref[1] + w2_ref[...]
    dis = dis_ref[...]
    h2 = dis * s2
    o2 = jnp.dot(h2, w21_ref[...], preferred_element_type=jnp.float32) + b21_ref[...]
    xa = jax.nn.relu(out0_ref[...])
    xb = jax.nn.relu(o1_ref[...])
    xc = jax.nn.relu(o2)
    dot = functools.partial(jnp.dot, preferred_element_type=jnp.float32)
    z0 = (dot(xa, w02_ref[0:512]) + dot(xb, w02_ref[512:1024])
          + dot(xc, w02_ref[1024:1536]) + b02_ref[...])
    y1 = (dot(xa, w12_ref[0:512]) + dot(xb, w12_ref[512:1024])
          + dot(xc, w12_ref[1024:1536]))
    z0_ref[...] = z0
    u2 = dis[:, :40] * y1
    u2_ref[...] = jnp.concatenate([u2, jnp.zeros((_RB, 88), jnp.float32)], axis=1)


def _tc_final_body(p_ref, u2_ref, dis_ref, z0_ref, b12_ref, out_ref):
    s3 = p_ref[0] + p_ref[1] + u2_ref[...]
    h3 = dis_ref[...][:, :40] * s3[:, :40]
    o12 = h3 + b12_ref[...]
    logits = jnp.concatenate([z0_ref[...], o12], axis=1)
    m = jnp.max(logits, axis=1, keepdims=True)
    ex = jnp.exp(logits - m)
    se = jnp.sum(ex, axis=1, keepdims=True)
    out_ref[...] = logits - m - jnp.log(se)


def _row_spec(width):
    return pl.BlockSpec((_RB, width), lambda i: (i, 0))


def _part_spec(width):
    return pl.BlockSpec((NC, _RB, width), lambda i: (0, i, 0))


def _full_spec(shape):
    nd = len(shape)
    return pl.BlockSpec(shape, lambda i: (0,) * nd)


def kernel(x, edge_index, W0_1, b0_1, W1_1, b1_1, W2_1, b2_1,
           W0_2, b0_2, W1_2, b1_2):
    E = edge_index.shape[1]
    assert E == NW * NCHUNK * K_EDGE

    row3 = edge_index[0].reshape(NW, NCHUNK, K_EDGE)
    col3 = edge_index[1].reshape(NW, NCHUNK, K_EDGE)

    deg_parts = _sc_degree(col3)
    u, dis, out0 = pl.pallas_call(
        _tc_prep_body,
        grid=(_GRID,),
        in_specs=[_part_spec(128), _row_spec(128),
                  _full_spec((128, 512)), _full_spec((1, 512))],
        out_specs=[_row_spec(128), _row_spec(128), _row_spec(512)],
        out_shape=[jax.ShapeDtypeStruct((N_NODES, 128), jnp.float32),
                   jax.ShapeDtypeStruct((N_NODES, 128), jnp.float32),
                   jax.ShapeDtypeStruct((N_NODES, 512), jnp.float32)],
    )(deg_parts, x, W0_1, b0_1.reshape(1, 512))

    p1 = _sc_scatter(u, row3, col3, 128)
    o1, w2 = pl.pallas_call(
        _tc_mid_body,
        grid=(_GRID,),
        in_specs=[_part_spec(128), _row_spec(128), _row_spec(128),
                  _full_spec((128, 512)), _full_spec((1, 512))],
        out_specs=[_row_spec(512), _row_spec(128)],
        out_shape=[jax.ShapeDtypeStruct((N_NODES, 512), jnp.float32),
                   jax.ShapeDtypeStruct((N_NODES, 128), jnp.float32)],
    )(p1, u, dis, W1_1, b1_1.reshape(1, 512))

    p2 = _sc_scatter(w2, row3, col3, 128)
    z0, u2 = pl.pallas_call(
        _tc_big_body,
        grid=(_GRID,),
        in_specs=[_part_spec(128), _row_spec(128), _row_spec(128),
                  _row_spec(512), _row_spec(512),
                  _full_spec((128, 512)), _full_spec((1, 512)),
                  _full_spec((1536, 40)), _full_spec((1, 40)),
                  _full_spec((1536, 40))],
        out_specs=[_row_spec(40), _row_spec(128)],
        out_shape=[jax.ShapeDtypeStruct((N_NODES, 40), jnp.float32),
                   jax.ShapeDtypeStruct((N_NODES, 128), jnp.float32)],
    )(p2, w2, dis, out0, o1, W2_1, b2_1.reshape(1, 512),
      W0_2, b0_2.reshape(1, 40), W1_2)

    p3 = _sc_scatter(u2, row3, col3, 128)
    out = pl.pallas_call(
        _tc_final_body,
        grid=(_GRID,),
        in_specs=[_part_spec(128), _row_spec(128), _row_spec(128),
                  _row_spec(40), _full_spec((1, 40))],
        out_specs=_row_spec(80),
        out_shape=jax.ShapeDtypeStruct((N_NODES, 80), jnp.float32),
    )(p3, u2, dis, z0, b1_2.reshape(1, 40))
    return out


# speedup vs baseline: 31.1545x; 1.0057x over previous
import functools

import jax
import jax.numpy as jnp
from jax import lax
from jax.experimental import pallas as pl
from jax.experimental.pallas import tpu as pltpu
from jax.experimental.pallas import tpu_sc as plsc

N_NODES = 10000
N_PAD = 10240
NC = 2
NS = 16
NW = NC * NS
K_EDGE = 125
NCHUNK = 80
HCHUNK = NCHUNK // 2
ROWS_SUB = N_PAD // NS
ZCH = 32
NBUF = 2
DWIN = 8

_MESH = plsc.VectorSubcoreMesh(core_axis_name="c", subcore_axis_name="s")


def _vfill(buf, rows, width, val):
    def body(i, carry):
        for j in range(width // 16):
            buf[i, pl.ds(j * 16, 16)] = jnp.full((16,), val, jnp.float32)
        return carry

    lax.fori_loop(0, rows, body, 0)


def _zero_stripe(acc, zbuf, s, width):
    _vfill(zbuf, ZCH, width, 0.0)
    for z in range(ROWS_SUB // ZCH):
        pltpu.sync_copy(zbuf, acc.at[pl.ds(s * ROWS_SUB + z * ZCH, ZCH)])


def _sc_degree_body(col_hbm, out_hbm, idx_v, ones_v, zbuf, acc, sem):
    c = lax.axis_index("c")
    s = lax.axis_index("s")
    wid = c * NS + s
    _zero_stripe(acc, zbuf, s, 128)
    _vfill(ones_v, K_EDGE, 128, 1.0)
    pltpu.sync_copy(col_hbm.at[wid], idx_v)
    plsc.subcore_barrier()

    def chunk(j, carry):
        @pl.when(j >= DWIN)
        def _drain():
            pltpu.make_async_copy(ones_v, acc.at[idx_v.at[j - DWIN]], sem).wait()

        pltpu.async_copy(ones_v, acc.at[idx_v.at[j]], sem, add=True)
        return carry

    lax.fori_loop(0, NCHUNK, chunk, 0)

    def drain(j, carry):
        pltpu.make_async_copy(ones_v, acc.at[idx_v.at[j]], sem).wait()
        return carry

    lax.fori_loop(NCHUNK - DWIN, NCHUNK, drain, 0)
    plsc.subcore_barrier()
    pltpu.sync_copy(acc.at[pl.ds(s * ROWS_SUB, ROWS_SUB)],
                    out_hbm.at[c, pl.ds(s * ROWS_SUB, ROWS_SUB)])


def _sc_scatter_body(vals_hbm, row_hbm, col_hbm, out_hbm,
                     ridx_v, cidx_v, gbuf, zbuf, acc,
                     gs0, gs1, ss0, ss1):
    gsems = (gs0, gs1)
    ssems = (ss0, ss1)
    c = lax.axis_index("c")
    s = lax.axis_index("s")
    wid = c * NS + s
    _zero_stripe(acc, zbuf, s, acc.shape[1])
    plsc.subcore_barrier()

    for h in range(2):
        pltpu.sync_copy(row_hbm.at[wid, pl.ds(h * HCHUNK, HCHUNK)], ridx_v)
        pltpu.sync_copy(col_hbm.at[wid, pl.ds(h * HCHUNK, HCHUNK)], cidx_v)
        for b in range(NBUF):
            pltpu.async_copy(vals_hbm.at[ridx_v.at[b]], gbuf.at[b], gsems[b])

        def round_(r, carry):
            for b in range(NBUF):
                j = r * NBUF + b
                pltpu.make_async_copy(vals_hbm.at[ridx_v.at[j]], gbuf.at[b],
                                      gsems[b]).wait()
                pltpu.async_copy(gbuf.at[b], acc.at[cidx_v.at[j]], ssems[b],
                                 add=True)
                pb = (b - 1) % NBUF
                jprev = j - 1
                jnext = jprev + NBUF

                @pl.when(jnp.logical_and(jprev >= 0, jnext < HCHUNK))
                def _refill():
                    pltpu.make_async_copy(gbuf.at[pb],
                                          acc.at[cidx_v.at[jprev]],
                                          ssems[pb]).wait()
                    pltpu.async_copy(vals_hbm.at[ridx_v.at[jnext]],
                                     gbuf.at[pb], gsems[pb])
            return carry

        lax.fori_loop(0, HCHUNK // NBUF, round_, 0)
        for b in range(NBUF):
            j = HCHUNK - NBUF + b
            pltpu.make_async_copy(gbuf.at[b], acc.at[cidx_v.at[j]],
                                  ssems[b]).wait()
    plsc.subcore_barrier()
    pltpu.sync_copy(acc.at[pl.ds(s * ROWS_SUB, ROWS_SUB)],
                    out_hbm.at[c, pl.ds(s * ROWS_SUB, ROWS_SUB)])


def _sc_degree(col3):
    call = pl.kernel(
        _sc_degree_body,
        out_type=jax.ShapeDtypeStruct((NC, N_PAD, 128), jnp.float32),
        mesh=_MESH,
        scratch_types=[
            pltpu.VMEM((NCHUNK, K_EDGE), jnp.int32),
            pltpu.VMEM((K_EDGE, 128), jnp.float32),
            pltpu.VMEM((ZCH, 128), jnp.float32),
            pltpu.VMEM_SHARED((N_PAD, 128), jnp.float32),
            pltpu.SemaphoreType.DMA,
        ],
    )
    return call(col3)


def _sc_scatter(vals, row3, col3, width):
    call = pl.kernel(
        _sc_scatter_body,
        out_type=jax.ShapeDtypeStruct((NC, N_PAD, width), jnp.float32),
        mesh=_MESH,
        scratch_types=[
            pltpu.VMEM((HCHUNK, K_EDGE), jnp.int32),
            pltpu.VMEM((HCHUNK, K_EDGE), jnp.int32),
            pltpu.VMEM((NBUF, K_EDGE, width), jnp.float32),
            pltpu.VMEM((ZCH, width), jnp.float32),
            pltpu.VMEM_SHARED((N_PAD, width), jnp.float32),
        ] + [pltpu.SemaphoreType.DMA] * (2 * NBUF),
    )
    return call(vals, row3, col3)


_RB = 400
_GRID = N_NODES // _RB


def _tc_prep_body(dp_ref, x_ref, w_ref, b_ref, u_ref, dis_ref, out0_ref):
    deg = dp_ref[0] + dp_ref[1] + 1.0
    disb = lax.rsqrt(deg)
    u_ref[...] = x_ref[...] * disb
    dis_ref[...] = disb
    out0_ref[...] = jnp.dot(x_ref[...], w_ref[...],
                            preferred_element_type=jnp.float32) + b_ref[...]


def _tc_w2_body(p_ref, u_ref, dis_ref, w2_ref):
    s1 = p_ref[0] + p_ref[1] + u_ref[...]
    dis = dis_ref[...]
    w2_ref[...] = dis * dis * s1


def _tc_o1_body(p_ref, u_ref, dis_ref, w_ref, b_ref, o1_ref):
    s1 = p_ref[0] + p_ref[1] + u_ref[...]
    h1 = dis_ref[...] * s1
    o1_ref[...] = jnp.dot(h1, w_ref[...],
                          preferred_element_type=jnp.float32) + b_ref[...]


def _tc_big_body(p_ref, w2_ref, dis_ref, out0_ref, o1_ref,
                 w21_ref, b21_ref, w02_ref, w12_ref,
                 zc_ref, u2_ref):
    s2 = p_ref[0] + p_ref[1] + w2_ref[...]
    dis = dis_ref[...]
    h2 = dis * s2
    o2 = jnp.dot(h2, w21_ref[...], preferred_element_type=jnp.float32) + b21_ref[...]
    xa = jax.nn.relu(out0_ref[...])
    xb = jax.nn.relu(o1_ref[...])
    xc = jax.nn.relu(o2)
    dot = functools.partial(jnp.dot, preferred_element_type=jnp.float32)
    y1 = (dot(xa, w12_ref[0:512]) + dot(xb, w12_ref[512:1024])
          + dot(xc, w12_ref[1024:1536]))
    zc_ref[...] = dot(xc, w02_ref[1024:1536])
    u2 = dis[:, :40] * y1
    u2_ref[...] = jnp.concatenate([u2, jnp.zeros((_RB, 88), jnp.float32)], axis=1)


def _tc_z0_body(out0_ref, o1_ref, zc_ref, w02_ref, b02_ref, z0_ref):
    xa = jax.nn.relu(out0_ref[...])
    xb = jax.nn.relu(o1_ref[...])
    dot = functools.partial(jnp.dot, preferred_element_type=jnp.float32)
    z0_ref[...] = (dot(xa, w02_ref[0:512]) + dot(xb, w02_ref[512:1024])
                   + zc_ref[...] + b02_ref[...])


def _tc_final_body(p_ref, u2_ref, dis_ref, z0_ref, b12_ref, out_ref):
    s3 = p_ref[0] + p_ref[1] + u2_ref[...]
    h3 = dis_ref[...][:, :40] * s3[:, :40]
    o12 = h3 + b12_ref[...]
    logits = jnp.concatenate([z0_ref[...], o12], axis=1)
    m = jnp.max(logits, axis=1, keepdims=True)
    ex = jnp.exp(logits - m)
    se = jnp.sum(ex, axis=1, keepdims=True)
    out_ref[...] = logits - m - jnp.log(se)


def _row_spec(width):
    return pl.BlockSpec((_RB, width), lambda i: (i, 0))


def _part_spec(width):
    return pl.BlockSpec((NC, _RB, width), lambda i: (0, i, 0))


def _full_spec(shape):
    nd = len(shape)
    return pl.BlockSpec(shape, lambda i: (0,) * nd)


def kernel(x, edge_index, W0_1, b0_1, W1_1, b1_1, W2_1, b2_1,
           W0_2, b0_2, W1_2, b1_2):
    E = edge_index.shape[1]
    assert E == NW * NCHUNK * K_EDGE

    row3 = edge_index[0].reshape(NW, NCHUNK, K_EDGE)
    col3 = edge_index[1].reshape(NW, NCHUNK, K_EDGE)

    deg_parts = _sc_degree(col3)
    u, dis, out0 = pl.pallas_call(
        _tc_prep_body,
        grid=(_GRID,),
        in_specs=[_part_spec(128), _row_spec(128),
                  _full_spec((128, 512)), _full_spec((1, 512))],
        out_specs=[_row_spec(128), _row_spec(128), _row_spec(512)],
        out_shape=[jax.ShapeDtypeStruct((N_NODES, 128), jnp.float32),
                   jax.ShapeDtypeStruct((N_NODES, 128), jnp.float32),
                   jax.ShapeDtypeStruct((N_NODES, 512), jnp.float32)],
    )(deg_parts, x, W0_1, b0_1.reshape(1, 512))

    p1 = _sc_scatter(u, row3, col3, 128)
    w2 = pl.pallas_call(
        _tc_w2_body,
        grid=(_GRID,),
        in_specs=[_part_spec(128), _row_spec(128), _row_spec(128)],
        out_specs=_row_spec(128),
        out_shape=jax.ShapeDtypeStruct((N_NODES, 128), jnp.float32),
    )(p1, u, dis)

    p2 = _sc_scatter(w2, row3, col3, 128)
    o1 = pl.pallas_call(
        _tc_o1_body,
        grid=(_GRID,),
        in_specs=[_part_spec(128), _row_spec(128), _row_spec(128),
                  _full_spec((128, 512)), _full_spec((1, 512))],
        out_specs=_row_spec(512),
        out_shape=jax.ShapeDtypeStruct((N_NODES, 512), jnp.float32),
    )(p1, u, dis, W1_1, b1_1.reshape(1, 512))

    zc, u2 = pl.pallas_call(
        _tc_big_body,
        grid=(_GRID,),
        in_specs=[_part_spec(128), _row_spec(128), _row_spec(128),
                  _row_spec(512), _row_spec(512),
                  _full_spec((128, 512)), _full_spec((1, 512)),
                  _full_spec((1536, 40)), _full_spec((1536, 40))],
        out_specs=[_row_spec(40), _row_spec(128)],
        out_shape=[jax.ShapeDtypeStruct((N_NODES, 40), jnp.float32),
                   jax.ShapeDtypeStruct((N_NODES, 128), jnp.float32)],
    )(p2, w2, dis, out0, o1, W2_1, b2_1.reshape(1, 512), W0_2, W1_2)

    p3 = _sc_scatter(u2, row3, col3, 128)
    z0 = pl.pallas_call(
        _tc_z0_body,
        grid=(_GRID,),
        in_specs=[_row_spec(512), _row_spec(512), _row_spec(40),
                  _full_spec((1536, 40)), _full_spec((1, 40))],
        out_specs=_row_spec(40),
        out_shape=jax.ShapeDtypeStruct((N_NODES, 40), jnp.float32),
    )(out0, o1, zc, W0_2, b0_2.reshape(1, 40))

    out = pl.pallas_call(
        _tc_final_body,
        grid=(_GRID,),
        in_specs=[_part_spec(128), _row_spec(128), _row_spec(128),
                  _row_spec(40), _full_spec((1, 40))],
        out_specs=_row_spec(80),
        out_shape=jax.ShapeDtypeStruct((N_NODES, 80), jnp.float32),
    )(p3, u2, dis, z0, b1_2.reshape(1, 40))
    return out


# speedup vs baseline: 31.5470x; 1.0126x over previous
import functools

import jax
import jax.numpy as jnp
from jax import lax
from jax.experimental import pallas as pl
from jax.experimental.pallas import tpu as pltpu
from jax.experimental.pallas import tpu_sc as plsc

N_NODES = 10000
N_PAD = 10240
NC = 2
NS = 16
NW = NC * NS
K_EDGE = 125
NCHUNK = 80
HCHUNK = NCHUNK // 2
ROWS_SUB = N_PAD // NS
ZCH = 32
NBUF = 2
DWIN = 8

_MESH = plsc.VectorSubcoreMesh(core_axis_name="c", subcore_axis_name="s")


def _vfill(buf, rows, width, val):
    def body(i, carry):
        for j in range(width // 16):
            buf[i, pl.ds(j * 16, 16)] = jnp.full((16,), val, jnp.float32)
        return carry

    lax.fori_loop(0, rows, body, 0)


def _zero_stripe(acc, zbuf, s, width):
    _vfill(zbuf, ZCH, width, 0.0)
    for z in range(ROWS_SUB // ZCH):
        pltpu.sync_copy(zbuf, acc.at[pl.ds(s * ROWS_SUB + z * ZCH, ZCH)])


def _sc_degree_body(col_hbm, out_hbm, idx_v, ones_v, zbuf, acc, sem):
    c = lax.axis_index("c")
    s = lax.axis_index("s")
    wid = c * NS + s
    _zero_stripe(acc, zbuf, s, 128)
    _vfill(ones_v, K_EDGE, 128, 1.0)
    pltpu.sync_copy(col_hbm.at[wid], idx_v)
    plsc.subcore_barrier()

    def chunk(j, carry):
        @pl.when(j >= DWIN)
        def _drain():
            pltpu.make_async_copy(ones_v, acc.at[idx_v.at[j - DWIN]], sem).wait()

        pltpu.async_copy(ones_v, acc.at[idx_v.at[j]], sem, add=True)
        return carry

    lax.fori_loop(0, NCHUNK, chunk, 0)

    def drain(j, carry):
        pltpu.make_async_copy(ones_v, acc.at[idx_v.at[j]], sem).wait()
        return carry

    lax.fori_loop(NCHUNK - DWIN, NCHUNK, drain, 0)
    plsc.subcore_barrier()
    pltpu.sync_copy(acc.at[pl.ds(s * ROWS_SUB, ROWS_SUB)],
                    out_hbm.at[c, pl.ds(s * ROWS_SUB, ROWS_SUB)])


def _sc_scatter_body(vals_hbm, row_hbm, col_hbm, out_hbm,
                     ridx_v, cidx_v, gbuf, zbuf, acc,
                     gs0, gs1, ss0, ss1):
    gsems = (gs0, gs1)
    ssems = (ss0, ss1)
    c = lax.axis_index("c")
    s = lax.axis_index("s")
    wid = c * NS + s
    _zero_stripe(acc, zbuf, s, acc.shape[1])
    plsc.subcore_barrier()

    for h in range(2):
        pltpu.sync_copy(row_hbm.at[wid, pl.ds(h * HCHUNK, HCHUNK)], ridx_v)
        pltpu.sync_copy(col_hbm.at[wid, pl.ds(h * HCHUNK, HCHUNK)], cidx_v)
        for b in range(NBUF):
            pltpu.async_copy(vals_hbm.at[ridx_v.at[b]], gbuf.at[b], gsems[b])

        def round_(r, carry):
            for b in range(NBUF):
                j = r * NBUF + b
                pltpu.make_async_copy(vals_hbm.at[ridx_v.at[j]], gbuf.at[b],
                                      gsems[b]).wait()
                pltpu.async_copy(gbuf.at[b], acc.at[cidx_v.at[j]], ssems[b],
                                 add=True)
                pb = (b - 1) % NBUF
                jprev = j - 1
                jnext = jprev + NBUF

                @pl.when(jnp.logical_and(jprev >= 0, jnext < HCHUNK))
                def _refill():
                    pltpu.make_async_copy(gbuf.at[pb],
                                          acc.at[cidx_v.at[jprev]],
                                          ssems[pb]).wait()
                    pltpu.async_copy(vals_hbm.at[ridx_v.at[jnext]],
                                     gbuf.at[pb], gsems[pb])
            return carry

        lax.fori_loop(0, HCHUNK // NBUF, round_, 0)
        for b in range(NBUF):
            j = HCHUNK - NBUF + b
            pltpu.make_async_copy(gbuf.at[b], acc.at[cidx_v.at[j]],
                                  ssems[b]).wait()
    plsc.subcore_barrier()
    pltpu.sync_copy(acc.at[pl.ds(s * ROWS_SUB, ROWS_SUB)],
                    out_hbm.at[c, pl.ds(s * ROWS_SUB, ROWS_SUB)])


def _sc_degree(col3):
    call = pl.kernel(
        _sc_degree_body,
        out_type=jax.ShapeDtypeStruct((NC, N_PAD, 128), jnp.float32),
        mesh=_MESH,
        scratch_types=[
            pltpu.VMEM((NCHUNK, K_EDGE), jnp.int32),
            pltpu.VMEM((K_EDGE, 128), jnp.float32),
            pltpu.VMEM((ZCH, 128), jnp.float32),
            pltpu.VMEM_SHARED((N_PAD, 128), jnp.float32),
            pltpu.SemaphoreType.DMA,
        ],
    )
    return call(col3)


def _sc_scatter(vals, row3, col3, width):
    call = pl.kernel(
        _sc_scatter_body,
        out_type=jax.ShapeDtypeStruct((NC, N_PAD, width), jnp.float32),
        mesh=_MESH,
        scratch_types=[
            pltpu.VMEM((HCHUNK, K_EDGE), jnp.int32),
            pltpu.VMEM((HCHUNK, K_EDGE), jnp.int32),
            pltpu.VMEM((NBUF, K_EDGE, width), jnp.float32),
            pltpu.VMEM((ZCH, width), jnp.float32),
            pltpu.VMEM_SHARED((N_PAD, width), jnp.float32),
        ] + [pltpu.SemaphoreType.DMA] * (2 * NBUF),
    )
    return call(vals, row3, col3)


_RB = 400
_GRID = N_NODES // _RB


def _tc_prep_body(dp_ref, x_ref, w_ref, b_ref, w02a_ref, w12a_ref,
                  u_ref, dis_ref, za_ref, ya_ref):
    deg = dp_ref[0] + dp_ref[1] + 1.0
    disb = lax.rsqrt(deg)
    x = x_ref[...]
    u_ref[...] = x * disb
    dis_ref[...] = disb
    xa = jax.nn.relu(jnp.dot(x, w_ref[...],
                             preferred_element_type=jnp.float32) + b_ref[...])
    za_ref[...] = jnp.dot(xa, w02a_ref[...], preferred_element_type=jnp.float32)
    ya_ref[...] = jnp.dot(xa, w12a_ref[...], preferred_element_type=jnp.float32)


def _tc_mid_body(p_ref, u_ref, dis_ref, w_ref, b_ref, w02b_ref, w12b_ref,
                 w2_ref, zb_ref, yb_ref):
    s1 = p_ref[0] + p_ref[1] + u_ref[...]
    dis = dis_ref[...]
    h1 = dis * s1
    w2_ref[...] = h1 * dis
    xb = jax.nn.relu(jnp.dot(h1, w_ref[...],
                             preferred_element_type=jnp.float32) + b_ref[...])
    zb_ref[...] = jnp.dot(xb, w02b_ref[...], preferred_element_type=jnp.float32)
    yb_ref[...] = jnp.dot(xb, w12b_ref[...], preferred_element_type=jnp.float32)


def _tc_big_body(p_ref, w2_ref, dis_ref, za_ref, ya_ref, zb_ref, yb_ref,
                 w21_ref, b21_ref, w02c_ref, b02_ref, w12c_ref,
                 z0_ref, u2_ref):
    s2 = p_ref[0] + p_ref[1] + w2_ref[...]
    dis = dis_ref[...]
    h2 = dis * s2
    o2 = jnp.dot(h2, w21_ref[...], preferred_element_type=jnp.float32) + b21_ref[...]
    xc = jax.nn.relu(o2)
    dot = functools.partial(jnp.dot, preferred_element_type=jnp.float32)
    z0_ref[...] = (za_ref[...] + zb_ref[...] + dot(xc, w02c_ref[...])
                   + b02_ref[...])
    y1 = ya_ref[...] + yb_ref[...] + dot(xc, w12c_ref[...])
    u2 = dis[:, :40] * y1
    u2_ref[...] = jnp.concatenate([u2, jnp.zeros((_RB, 88), jnp.float32)], axis=1)


def _tc_final_body(p_ref, u2_ref, dis_ref, z0_ref, b12_ref, out_ref):
    s3 = p_ref[0] + p_ref[1] + u2_ref[...]
    h3 = dis_ref[...][:, :40] * s3[:, :40]
    o12 = h3 + b12_ref[...]
    logits = jnp.concatenate([z0_ref[...], o12], axis=1)
    m = jnp.max(logits, axis=1, keepdims=True)
    ex = jnp.exp(logits - m)
    se = jnp.sum(ex, axis=1, keepdims=True)
    out_ref[...] = logits - m - jnp.log(se)


def _row_spec(width):
    return pl.BlockSpec((_RB, width), lambda i: (i, 0))


def _part_spec(width):
    return pl.BlockSpec((NC, _RB, width), lambda i: (0, i, 0))


def _full_spec(shape):
    nd = len(shape)
    return pl.BlockSpec(shape, lambda i: (0,) * nd)


def kernel(x, edge_index, W0_1, b0_1, W1_1, b1_1, W2_1, b2_1,
           W0_2, b0_2, W1_2, b1_2):
    E = edge_index.shape[1]
    assert E == NW * NCHUNK * K_EDGE

    row3 = edge_index[0].reshape(NW, NCHUNK, K_EDGE)
    col3 = edge_index[1].reshape(NW, NCHUNK, K_EDGE)

    deg_parts = _sc_degree(col3)
    u, dis, za, ya = pl.pallas_call(
        _tc_prep_body,
        grid=(_GRID,),
        in_specs=[_part_spec(128), _row_spec(128),
                  _full_spec((128, 512)), _full_spec((1, 512)),
                  _full_spec((512, 40)), _full_spec((512, 40))],
        out_specs=[_row_spec(128), _row_spec(128), _row_spec(40),
                   _row_spec(40)],
        out_shape=[jax.ShapeDtypeStruct((N_NODES, 128), jnp.float32),
                   jax.ShapeDtypeStruct((N_NODES, 128), jnp.float32),
                   jax.ShapeDtypeStruct((N_NODES, 40), jnp.float32),
                   jax.ShapeDtypeStruct((N_NODES, 40), jnp.float32)],
    )(deg_parts, x, W0_1, b0_1.reshape(1, 512), W0_2[0:512], W1_2[0:512])

    p1 = _sc_scatter(u, row3, col3, 128)
    w2, zb, yb = pl.pallas_call(
        _tc_mid_body,
        grid=(_GRID,),
        in_specs=[_part_spec(128), _row_spec(128), _row_spec(128),
                  _full_spec((128, 512)), _full_spec((1, 512)),
                  _full_spec((512, 40)), _full_spec((512, 40))],
        out_specs=[_row_spec(128), _row_spec(40), _row_spec(40)],
        out_shape=[jax.ShapeDtypeStruct((N_NODES, 128), jnp.float32),
                   jax.ShapeDtypeStruct((N_NODES, 40), jnp.float32),
                   jax.ShapeDtypeStruct((N_NODES, 40), jnp.float32)],
    )(p1, u, dis, W1_1, b1_1.reshape(1, 512),
      W0_2[512:1024], W1_2[512:1024])

    p2 = _sc_scatter(w2, row3, col3, 128)
    z0, u2 = pl.pallas_call(
        _tc_big_body,
        grid=(_GRID,),
        in_specs=[_part_spec(128), _row_spec(128), _row_spec(128),
                  _row_spec(40), _row_spec(40), _row_spec(40), _row_spec(40),
                  _full_spec((128, 512)), _full_spec((1, 512)),
                  _full_spec((512, 40)), _full_spec((1, 40)),
                  _full_spec((512, 40))],
        out_specs=[_row_spec(40), _row_spec(128)],
        out_shape=[jax.ShapeDtypeStruct((N_NODES, 40), jnp.float32),
                   jax.ShapeDtypeStruct((N_NODES, 128), jnp.float32)],
    )(p2, w2, dis, za, ya, zb, yb, W2_1, b2_1.reshape(1, 512),
      W0_2[1024:1536], b0_2.reshape(1, 40), W1_2[1024:1536])

    p3 = _sc_scatter(u2, row3, col3, 128)
    out = pl.pallas_call(
        _tc_final_body,
        grid=(_GRID,),
        in_specs=[_part_spec(128), _row_spec(128), _row_spec(128),
                  _row_spec(40), _full_spec((1, 40))],
        out_specs=_row_spec(80),
        out_shape=jax.ShapeDtypeStruct((N_NODES, 80), jnp.float32),
    )(p3, u2, dis, z0, b1_2.reshape(1, 40))
    return out


# speedup vs baseline: 32.1725x; 1.0198x over previous
import functools

import jax
import jax.numpy as jnp
from jax import lax
from jax.experimental import pallas as pl
from jax.experimental.pallas import tpu as pltpu
from jax.experimental.pallas import tpu_sc as plsc

N_NODES = 10000
N_PAD = 10240
NC = 2
NS = 16
NW = NC * NS
K_EDGE = 125
NCHUNK = 80
HCHUNK = NCHUNK // 2
ROWS_SUB = N_PAD // NS
ZCH = 32
NBUF = 2
DWIN = 8

_MESH = plsc.VectorSubcoreMesh(core_axis_name="c", subcore_axis_name="s")


def _vfill(buf, rows, width, val):
    def body(i, carry):
        for j in range(width // 16):
            buf[i, pl.ds(j * 16, 16)] = jnp.full((16,), val, jnp.float32)
        return carry

    lax.fori_loop(0, rows, body, 0)


def _zero_stripe_start(acc, zbuf, s, width, sem):
    _vfill(zbuf, ZCH, width, 0.0)
    for z in range(ROWS_SUB // ZCH):
        pltpu.async_copy(zbuf, acc.at[pl.ds(s * ROWS_SUB + z * ZCH, ZCH)], sem)


def _zero_stripe_drain(acc, zbuf, s, sem):
    for z in range(ROWS_SUB // ZCH):
        pltpu.make_async_copy(zbuf, acc.at[pl.ds(s * ROWS_SUB + z * ZCH, ZCH)],
                              sem).wait()


def _sc_degree_body(col_hbm, out_hbm, idx_v, ones_v, zbuf, acc, sem):
    c = lax.axis_index("c")
    s = lax.axis_index("s")
    wid = c * NS + s
    _zero_stripe_start(acc, zbuf, s, 128, sem)
    _vfill(ones_v, K_EDGE, 128, 1.0)
    pltpu.sync_copy(col_hbm.at[wid], idx_v)
    _zero_stripe_drain(acc, zbuf, s, sem)
    plsc.subcore_barrier()

    def chunk(j, carry):
        @pl.when(j >= DWIN)
        def _drain():
            pltpu.make_async_copy(ones_v, acc.at[idx_v.at[j - DWIN]], sem).wait()

        pltpu.async_copy(ones_v, acc.at[idx_v.at[j]], sem, add=True)
        return carry

    lax.fori_loop(0, NCHUNK, chunk, 0)

    def drain(j, carry):
        pltpu.make_async_copy(ones_v, acc.at[idx_v.at[j]], sem).wait()
        return carry

    lax.fori_loop(NCHUNK - DWIN, NCHUNK, drain, 0)
    plsc.subcore_barrier()
    pltpu.sync_copy(acc.at[pl.ds(s * ROWS_SUB, ROWS_SUB)],
                    out_hbm.at[c, pl.ds(s * ROWS_SUB, ROWS_SUB)])


def _sc_scatter_body(vals_hbm, row_hbm, col_hbm, out_hbm,
                     ridx_v, cidx_v, gbuf, zbuf, acc,
                     gs0, gs1, ss0, ss1):
    gsems = (gs0, gs1)
    ssems = (ss0, ss1)
    c = lax.axis_index("c")
    s = lax.axis_index("s")
    wid = c * NS + s
    _zero_stripe_start(acc, zbuf, s, acc.shape[1], ss0)

    for h in range(2):
        pltpu.sync_copy(row_hbm.at[wid, pl.ds(h * HCHUNK, HCHUNK)], ridx_v)
        pltpu.sync_copy(col_hbm.at[wid, pl.ds(h * HCHUNK, HCHUNK)], cidx_v)
        for b in range(NBUF):
            pltpu.async_copy(vals_hbm.at[ridx_v.at[b]], gbuf.at[b], gsems[b])
        if h == 0:
            _zero_stripe_drain(acc, zbuf, s, ss0)
            plsc.subcore_barrier()

        def round_(r, carry):
            for b in range(NBUF):
                j = r * NBUF + b
                pltpu.make_async_copy(vals_hbm.at[ridx_v.at[j]], gbuf.at[b],
                                      gsems[b]).wait()
                pltpu.async_copy(gbuf.at[b], acc.at[cidx_v.at[j]], ssems[b],
                                 add=True)
                pb = (b - 1) % NBUF
                jprev = j - 1
                jnext = jprev + NBUF

                @pl.when(jnp.logical_and(jprev >= 0, jnext < HCHUNK))
                def _refill():
                    pltpu.make_async_copy(gbuf.at[pb],
                                          acc.at[cidx_v.at[jprev]],
                                          ssems[pb]).wait()
                    pltpu.async_copy(vals_hbm.at[ridx_v.at[jnext]],
                                     gbuf.at[pb], gsems[pb])
            return carry

        lax.fori_loop(0, HCHUNK // NBUF, round_, 0)
        for b in range(NBUF):
            j = HCHUNK - NBUF + b
            pltpu.make_async_copy(gbuf.at[b], acc.at[cidx_v.at[j]],
                                  ssems[b]).wait()
    plsc.subcore_barrier()
    pltpu.sync_copy(acc.at[pl.ds(s * ROWS_SUB, ROWS_SUB)],
                    out_hbm.at[c, pl.ds(s * ROWS_SUB, ROWS_SUB)])


def _sc_degree(col3):
    call = pl.kernel(
        _sc_degree_body,
        out_type=jax.ShapeDtypeStruct((NC, N_PAD, 128), jnp.float32),
        mesh=_MESH,
        scratch_types=[
            pltpu.VMEM((NCHUNK, K_EDGE), jnp.int32),
            pltpu.VMEM((K_EDGE, 128), jnp.float32),
            pltpu.VMEM((ZCH, 128), jnp.float32),
            pltpu.VMEM_SHARED((N_PAD, 128), jnp.float32),
            pltpu.SemaphoreType.DMA,
        ],
    )
    return call(col3)


def _sc_scatter(vals, row3, col3, width):
    call = pl.kernel(
        _sc_scatter_body,
        out_type=jax.ShapeDtypeStruct((NC, N_PAD, width), jnp.float32),
        mesh=_MESH,
        scratch_types=[
            pltpu.VMEM((HCHUNK, K_EDGE), jnp.int32),
            pltpu.VMEM((HCHUNK, K_EDGE), jnp.int32),
            pltpu.VMEM((NBUF, K_EDGE, width), jnp.float32),
            pltpu.VMEM((ZCH, width), jnp.float32),
            pltpu.VMEM_SHARED((N_PAD, width), jnp.float32),
        ] + [pltpu.SemaphoreType.DMA] * (2 * NBUF),
    )
    return call(vals, row3, col3)


_RB = 400
_GRID = N_NODES // _RB


def _tc_prep_body(dp_ref, x_ref, w_ref, b_ref, w02a_ref, w12a_ref,
                  u_ref, dis_ref, za_ref, ya_ref):
    deg = dp_ref[0] + dp_ref[1] + 1.0
    disb = lax.rsqrt(deg)
    x = x_ref[...]
    u_ref[...] = x * disb
    dis_ref[...] = disb
    xa = jax.nn.relu(jnp.dot(x, w_ref[...],
                             preferred_element_type=jnp.float32) + b_ref[...])
    za_ref[...] = jnp.dot(xa, w02a_ref[...], preferred_element_type=jnp.float32)
    ya_ref[...] = jnp.dot(xa, w12a_ref[...], preferred_element_type=jnp.float32)


def _tc_mid_body(p_ref, u_ref, dis_ref, w_ref, b_ref, w02b_ref, w12b_ref,
                 w2_ref, zb_ref, yb_ref):
    s1 = p_ref[0] + p_ref[1] + u_ref[...]
    dis = dis_ref[...]
    h1 = dis * s1
    w2_ref[...] = h1 * dis
    xb = jax.nn.relu(jnp.dot(h1, w_ref[...],
                             preferred_element_type=jnp.float32) + b_ref[...])
    zb_ref[...] = jnp.dot(xb, w02b_ref[...], preferred_element_type=jnp.float32)
    yb_ref[...] = jnp.dot(xb, w12b_ref[...], preferred_element_type=jnp.float32)


def _tc_big_body(p_ref, w2_ref, dis_ref, za_ref, ya_ref, zb_ref, yb_ref,
                 w21_ref, b21_ref, w02c_ref, b02_ref, w12c_ref,
                 z0_ref, u2_ref):
    s2 = p_ref[0] + p_ref[1] + w2_ref[...]
    dis = dis_ref[...]
    h2 = dis * s2
    o2 = jnp.dot(h2, w21_ref[...], preferred_element_type=jnp.float32) + b21_ref[...]
    xc = jax.nn.relu(o2)
    dot = functools.partial(jnp.dot, preferred_element_type=jnp.float32)
    z0_ref[...] = (za_ref[...] + zb_ref[...] + dot(xc, w02c_ref[...])
                   + b02_ref[...])
    y1 = ya_ref[...] + yb_ref[...] + dot(xc, w12c_ref[...])
    u2 = dis[:, :40] * y1
    u2_ref[...] = jnp.concatenate([u2, jnp.zeros((_RB, 88), jnp.float32)], axis=1)


def _tc_final_body(p_ref, u2_ref, dis_ref, z0_ref, b12_ref, out_ref):
    s3 = p_ref[0] + p_ref[1] + u2_ref[...]
    h3 = dis_ref[...][:, :40] * s3[:, :40]
    o12 = h3 + b12_ref[...]
    logits = jnp.concatenate([z0_ref[...], o12], axis=1)
    m = jnp.max(logits, axis=1, keepdims=True)
    ex = jnp.exp(logits - m)
    se = jnp.sum(ex, axis=1, keepdims=True)
    out_ref[...] = logits - m - jnp.log(se)


def _row_spec(width):
    return pl.BlockSpec((_RB, width), lambda i: (i, 0))


def _part_spec(width):
    return pl.BlockSpec((NC, _RB, width), lambda i: (0, i, 0))


def _full_spec(shape):
    nd = len(shape)
    return pl.BlockSpec(shape, lambda i: (0,) * nd)


def kernel(x, edge_index, W0_1, b0_1, W1_1, b1_1, W2_1, b2_1,
           W0_2, b0_2, W1_2, b1_2):
    E = edge_index.shape[1]
    assert E == NW * NCHUNK * K_EDGE

    row3 = edge_index[0].reshape(NW, NCHUNK, K_EDGE)
    col3 = edge_index[1].reshape(NW, NCHUNK, K_EDGE)

    deg_parts = _sc_degree(col3)
    u, dis, za, ya = pl.pallas_call(
        _tc_prep_body,
        grid=(_GRID,),
        in_specs=[_part_spec(128), _row_spec(128),
                  _full_spec((128, 512)), _full_spec((1, 512)),
                  _full_spec((512, 40)), _full_spec((512, 40))],
        out_specs=[_row_spec(128), _row_spec(128), _row_spec(40),
                   _row_spec(40)],
        out_shape=[jax.ShapeDtypeStruct((N_NODES, 128), jnp.float32),
                   jax.ShapeDtypeStruct((N_NODES, 128), jnp.float32),
                   jax.ShapeDtypeStruct((N_NODES, 40), jnp.float32),
                   jax.ShapeDtypeStruct((N_NODES, 40), jnp.float32)],
    )(deg_parts, x, W0_1, b0_1.reshape(1, 512), W0_2[0:512], W1_2[0:512])

    p1 = _sc_scatter(u, row3, col3, 128)
    w2, zb, yb = pl.pallas_call(
        _tc_mid_body,
        grid=(_GRID,),
        in_specs=[_part_spec(128), _row_spec(128), _row_spec(128),
                  _full_spec((128, 512)), _full_spec((1, 512)),
                  _full_spec((512, 40)), _full_spec((512, 40))],
        out_specs=[_row_spec(128), _row_spec(40), _row_spec(40)],
        out_shape=[jax.ShapeDtypeStruct((N_NODES, 128), jnp.float32),
                   jax.ShapeDtypeStruct((N_NODES, 40), jnp.float32),
                   jax.ShapeDtypeStruct((N_NODES, 40), jnp.float32)],
    )(p1, u, dis, W1_1, b1_1.reshape(1, 512),
      W0_2[512:1024], W1_2[512:1024])

    p2 = _sc_scatter(w2, row3, col3, 128)
    z0, u2 = pl.pallas_call(
        _tc_big_body,
        grid=(_GRID,),
        in_specs=[_part_spec(128), _row_spec(128), _row_spec(128),
                  _row_spec(40), _row_spec(40), _row_spec(40), _row_spec(40),
                  _full_spec((128, 512)), _full_spec((1, 512)),
                  _full_spec((512, 40)), _full_spec((1, 40)),
                  _full_spec((512, 40))],
        out_specs=[_row_spec(40), _row_spec(128)],
        out_shape=[jax.ShapeDtypeStruct((N_NODES, 40), jnp.float32),
                   jax.ShapeDtypeStruct((N_NODES, 128), jnp.float32)],
    )(p2, w2, dis, za, ya, zb, yb, W2_1, b2_1.reshape(1, 512),
      W0_2[1024:1536], b0_2.reshape(1, 40), W1_2[1024:1536])

    p3 = _sc_scatter(u2, row3, col3, 128)
    out = pl.pallas_call(
        _tc_final_body,
        grid=(_GRID,),
        in_specs=[_part_spec(128), _row_spec(128), _row_spec(128),
                  _row_spec(40), _full_spec((1, 40))],
        out_specs=_row_spec(80),
        out_shape=jax.ShapeDtypeStruct((N_NODES, 80), jnp.float32),
    )(p3, u2, dis, z0, b1_2.reshape(1, 40))
    return out


# speedup vs baseline: 32.3427x; 1.0053x over previous
import functools

import jax
import jax.numpy as jnp
from jax import lax
from jax.experimental import pallas as pl
from jax.experimental.pallas import tpu as pltpu
from jax.experimental.pallas import tpu_sc as plsc

N_NODES = 10000
N_PAD = 10240
NC = 2
NS = 16
NW = NC * NS
K_EDGE = 125
NCHUNK = 80
HCHUNK = NCHUNK // 2
ROWS_SUB = N_PAD // NS
ZCH = 32
NBUF = 2
DWIN = 16

_MESH = plsc.VectorSubcoreMesh(core_axis_name="c", subcore_axis_name="s")


def _vfill(buf, rows, width, val):
    def body(i, carry):
        for j in range(width // 16):
            buf[i, pl.ds(j * 16, 16)] = jnp.full((16,), val, jnp.float32)
        return carry

    lax.fori_loop(0, rows, body, 0)


def _zero_stripe_start(acc, zbuf, s, width, sem):
    _vfill(zbuf, ZCH, width, 0.0)
    for z in range(ROWS_SUB // ZCH):
        pltpu.async_copy(zbuf, acc.at[pl.ds(s * ROWS_SUB + z * ZCH, ZCH)], sem)


def _zero_stripe_drain(acc, zbuf, s, sem):
    for z in range(ROWS_SUB // ZCH):
        pltpu.make_async_copy(zbuf, acc.at[pl.ds(s * ROWS_SUB + z * ZCH, ZCH)],
                              sem).wait()


def _sc_degree_body(col_hbm, out_hbm, idx_v, ones_v, zbuf, acc, sem):
    c = lax.axis_index("c")
    s = lax.axis_index("s")
    wid = c * NS + s
    _zero_stripe_start(acc, zbuf, s, 128, sem)
    _vfill(ones_v, K_EDGE, 128, 1.0)
    pltpu.sync_copy(col_hbm.at[wid], idx_v)
    _zero_stripe_drain(acc, zbuf, s, sem)
    plsc.subcore_barrier()

    def chunk(j, carry):
        @pl.when(j >= DWIN)
        def _drain():
            pltpu.make_async_copy(ones_v, acc.at[idx_v.at[j - DWIN]], sem).wait()

        pltpu.async_copy(ones_v, acc.at[idx_v.at[j]], sem, add=True)
        return carry

    lax.fori_loop(0, NCHUNK, chunk, 0)

    def drain(j, carry):
        pltpu.make_async_copy(ones_v, acc.at[idx_v.at[j]], sem).wait()
        return carry

    lax.fori_loop(NCHUNK - DWIN, NCHUNK, drain, 0)
    plsc.subcore_barrier()
    pltpu.sync_copy(acc.at[pl.ds(s * ROWS_SUB, ROWS_SUB)],
                    out_hbm.at[c, pl.ds(s * ROWS_SUB, ROWS_SUB)])


def _sc_scatter_body(vals_hbm, row_hbm, col_hbm, out_hbm,
                     ridx_v, cidx_v, gbuf, zbuf, acc,
                     gs0, gs1, ss0, ss1):
    gsems = (gs0, gs1)
    ssems = (ss0, ss1)
    c = lax.axis_index("c")
    s = lax.axis_index("s")
    wid = c * NS + s
    _zero_stripe_start(acc, zbuf, s, acc.shape[1], ss0)

    for h in range(2):
        pltpu.sync_copy(row_hbm.at[wid, pl.ds(h * HCHUNK, HCHUNK)], ridx_v)
        pltpu.sync_copy(col_hbm.at[wid, pl.ds(h * HCHUNK, HCHUNK)], cidx_v)
        for b in range(NBUF):
            pltpu.async_copy(vals_hbm.at[ridx_v.at[b]], gbuf.at[b], gsems[b])
        if h == 0:
            _zero_stripe_drain(acc, zbuf, s, ss0)
            plsc.subcore_barrier()

        def round_(r, carry):
            for b in range(NBUF):
                j = r * NBUF + b
                pltpu.make_async_copy(vals_hbm.at[ridx_v.at[j]], gbuf.at[b],
                                      gsems[b]).wait()
                pltpu.async_copy(gbuf.at[b], acc.at[cidx_v.at[j]], ssems[b],
                                 add=True)
                pb = (b - 1) % NBUF
                jprev = j - 1
                jnext = jprev + NBUF

                @pl.when(jnp.logical_and(jprev >= 0, jnext < HCHUNK))
                def _refill():
                    pltpu.make_async_copy(gbuf.at[pb],
                                          acc.at[cidx_v.at[jprev]],
                                          ssems[pb]).wait()
                    pltpu.async_copy(vals_hbm.at[ridx_v.at[jnext]],
                                     gbuf.at[pb], gsems[pb])
            return carry

        lax.fori_loop(0, HCHUNK // NBUF, round_, 0)
        for b in range(NBUF):
            j = HCHUNK - NBUF + b
            pltpu.make_async_copy(gbuf.at[b], acc.at[cidx_v.at[j]],
                                  ssems[b]).wait()
    plsc.subcore_barrier()
    pltpu.sync_copy(acc.at[pl.ds(s * ROWS_SUB, ROWS_SUB)],
                    out_hbm.at[c, pl.ds(s * ROWS_SUB, ROWS_SUB)])


def _sc_degree(col3):
    call = pl.kernel(
        _sc_degree_body,
        out_type=jax.ShapeDtypeStruct((NC, N_PAD, 128), jnp.float32),
        mesh=_MESH,
        scratch_types=[
            pltpu.VMEM((NCHUNK, K_EDGE), jnp.int32),
            pltpu.VMEM((K_EDGE, 128), jnp.float32),
            pltpu.VMEM((ZCH, 128), jnp.float32),
            pltpu.VMEM_SHARED((N_PAD, 128), jnp.float32),
            pltpu.SemaphoreType.DMA,
        ],
    )
    return call(col3)


def _sc_scatter(vals, row3, col3, width):
    call = pl.kernel(
        _sc_scatter_body,
        out_type=jax.ShapeDtypeStruct((NC, N_PAD, width), jnp.float32),
        mesh=_MESH,
        scratch_types=[
            pltpu.VMEM((HCHUNK, K_EDGE), jnp.int32),
            pltpu.VMEM((HCHUNK, K_EDGE), jnp.int32),
            pltpu.VMEM((NBUF, K_EDGE, width), jnp.float32),
            pltpu.VMEM((ZCH, width), jnp.float32),
            pltpu.VMEM_SHARED((N_PAD, width), jnp.float32),
        ] + [pltpu.SemaphoreType.DMA] * (2 * NBUF),
    )
    return call(vals, row3, col3)


_RB = 400
_GRID = N_NODES // _RB


def _tc_prep_body(dp_ref, x_ref, w_ref, b_ref, w02a_ref, w12a_ref,
                  u_ref, dis_ref, za_ref, ya_ref):
    deg = dp_ref[0] + dp_ref[1] + 1.0
    disb = lax.rsqrt(deg)
    x = x_ref[...]
    u_ref[...] = x * disb
    dis_ref[...] = disb
    xa = jax.nn.relu(jnp.dot(x, w_ref[...],
                             preferred_element_type=jnp.float32) + b_ref[...])
    za_ref[...] = jnp.dot(xa, w02a_ref[...], preferred_element_type=jnp.float32)
    ya_ref[...] = jnp.dot(xa, w12a_ref[...], preferred_element_type=jnp.float32)


def _tc_mid_body(p_ref, u_ref, dis_ref, w_ref, b_ref, w02b_ref, w12b_ref,
                 w2_ref, zb_ref, yb_ref):
    s1 = p_ref[0] + p_ref[1] + u_ref[...]
    dis = dis_ref[...]
    h1 = dis * s1
    w2_ref[...] = h1 * dis
    xb = jax.nn.relu(jnp.dot(h1, w_ref[...],
                             preferred_element_type=jnp.float32) + b_ref[...])
    zb_ref[...] = jnp.dot(xb, w02b_ref[...], preferred_element_type=jnp.float32)
    yb_ref[...] = jnp.dot(xb, w12b_ref[...], preferred_element_type=jnp.float32)


def _tc_big_body(p_ref, w2_ref, dis_ref, za_ref, ya_ref, zb_ref, yb_ref,
                 w21_ref, b21_ref, w02c_ref, b02_ref, w12c_ref,
                 z0_ref, u2_ref):
    s2 = p_ref[0] + p_ref[1] + w2_ref[...]
    dis = dis_ref[...]
    h2 = dis * s2
    o2 = jnp.dot(h2, w21_ref[...], preferred_element_type=jnp.float32) + b21_ref[...]
    xc = jax.nn.relu(o2)
    dot = functools.partial(jnp.dot, preferred_element_type=jnp.float32)
    z0_ref[...] = (za_ref[...] + zb_ref[...] + dot(xc, w02c_ref[...])
                   + b02_ref[...])
    y1 = ya_ref[...] + yb_ref[...] + dot(xc, w12c_ref[...])
    u2 = dis[:, :40] * y1
    u2_ref[...] = jnp.concatenate([u2, jnp.zeros((_RB, 88), jnp.float32)], axis=1)


def _tc_final_body(p_ref, u2_ref, dis_ref, z0_ref, b12_ref, out_ref):
    s3 = p_ref[0] + p_ref[1] + u2_ref[...]
    h3 = dis_ref[...][:, :40] * s3[:, :40]
    o12 = h3 + b12_ref[...]
    logits = jnp.concatenate([z0_ref[...], o12], axis=1)
    m = jnp.max(logits, axis=1, keepdims=True)
    ex = jnp.exp(logits - m)
    se = jnp.sum(ex, axis=1, keepdims=True)
    out_ref[...] = logits - m - jnp.log(se)


def _row_spec(width):
    return pl.BlockSpec((_RB, width), lambda i: (i, 0))


def _part_spec(width):
    return pl.BlockSpec((NC, _RB, width), lambda i: (0, i, 0))


def _full_spec(shape):
    nd = len(shape)
    return pl.BlockSpec(shape, lambda i: (0,) * nd)


def kernel(x, edge_index, W0_1, b0_1, W1_1, b1_1, W2_1, b2_1,
           W0_2, b0_2, W1_2, b1_2):
    E = edge_index.shape[1]
    assert E == NW * NCHUNK * K_EDGE

    row3 = edge_index[0].reshape(NW, NCHUNK, K_EDGE)
    col3 = edge_index[1].reshape(NW, NCHUNK, K_EDGE)

    deg_parts = _sc_degree(col3)
    u, dis, za, ya = pl.pallas_call(
        _tc_prep_body,
        grid=(_GRID,),
        in_specs=[_part_spec(128), _row_spec(128),
                  _full_spec((128, 512)), _full_spec((1, 512)),
                  _full_spec((512, 40)), _full_spec((512, 40))],
        out_specs=[_row_spec(128), _row_spec(128), _row_spec(40),
                   _row_spec(40)],
        out_shape=[jax.ShapeDtypeStruct((N_NODES, 128), jnp.float32),
                   jax.ShapeDtypeStruct((N_NODES, 128), jnp.float32),
                   jax.ShapeDtypeStruct((N_NODES, 40), jnp.float32),
                   jax.ShapeDtypeStruct((N_NODES, 40), jnp.float32)],
    )(deg_parts, x, W0_1, b0_1.reshape(1, 512), W0_2[0:512], W1_2[0:512])

    p1 = _sc_scatter(u, row3, col3, 128)
    w2, zb, yb = pl.pallas_call(
        _tc_mid_body,
        grid=(_GRID,),
        in_specs=[_part_spec(128), _row_spec(128), _row_spec(128),
                  _full_spec((128, 512)), _full_spec((1, 512)),
                  _full_spec((512, 40)), _full_spec((512, 40))],
        out_specs=[_row_spec(128), _row_spec(40), _row_spec(40)],
        out_shape=[jax.ShapeDtypeStruct((N_NODES, 128), jnp.float32),
                   jax.ShapeDtypeStruct((N_NODES, 40), jnp.float32),
                   jax.ShapeDtypeStruct((N_NODES, 40), jnp.float32)],
    )(p1, u, dis, W1_1, b1_1.reshape(1, 512),
      W0_2[512:1024], W1_2[512:1024])

    p2 = _sc_scatter(w2, row3, col3, 128)
    z0, u2 = pl.pallas_call(
        _tc_big_body,
        grid=(_GRID,),
        in_specs=[_part_spec(128), _row_spec(128), _row_spec(128),
                  _row_spec(40), _row_spec(40), _row_spec(40), _row_spec(40),
                  _full_spec((128, 512)), _full_spec((1, 512)),
                  _full_spec((512, 40)), _full_spec((1, 40)),
                  _full_spec((512, 40))],
        out_specs=[_row_spec(40), _row_spec(128)],
        out_shape=[jax.ShapeDtypeStruct((N_NODES, 40), jnp.float32),
                   jax.ShapeDtypeStruct((N_NODES, 128), jnp.float32)],
    )(p2, w2, dis, za, ya, zb, yb, W2_1, b2_1.reshape(1, 512),
      W0_2[1024:1536], b0_2.reshape(1, 40), W1_2[1024:1536])

    p3 = _sc_scatter(u2, row3, col3, 128)
    out = pl.pallas_call(
        _tc_final_body,
        grid=(_GRID,),
        in_specs=[_part_spec(128), _row_spec(128), _row_spec(128),
                  _row_spec(40), _full_spec((1, 40))],
        out_specs=_row_spec(80),
        out_shape=jax.ShapeDtypeStruct((N_NODES, 80), jnp.float32),
    )(p3, u2, dis, z0, b1_2.reshape(1, 40))
    return out
